# baseline (speedup 1.0000x reference)
import functools

import jax
import jax.numpy as jnp
from jax import lax
from jax.experimental import pallas as pl
from jax.experimental.pallas import tpu as pltpu
from jax.experimental.pallas import tpu_sc as plsc

MAX_INT = 15.0
RTNE_MAGIC = 12582912.0
NW = 32
CHUNK = 128

_GDN = lax.GatherDimensionNumbers(
    offset_dims=(), collapsed_slice_dims=(0,), start_index_map=(0,)
)


def _permute(v, idx):
    return lax.gather(
        v,
        idx[:, None],
        dimension_numbers=_GDN,
        slice_sizes=(1,),
        mode=lax.GatherScatterMode.PROMISE_IN_BOUNDS,
    )


def _sc_body(x_hbm, o_hbm, in_v, out_v, sem_in, sem_out):
    wid = lax.axis_index("s") * 2 + lax.axis_index("c")
    total_rows = x_hbm.shape[0]
    rows_per_w = total_rows // NW
    base = wid * rows_per_w
    n_chunks = rows_per_w // CHUNK
    lanes = lax.iota(jnp.int32, 16)

    def chunk_body(ci, carry):
        start = base + ci * CHUNK
        pltpu.sync_copy(x_hbm.at[pl.ds(start, CHUNK)], in_v)

        def row_body(r, carry2):
            vs = [in_v[r, k * 16:(k + 1) * 16] for k in range(8)]
            mnv = vs[0]
            mxv = vs[0]
            for k in range(1, 8):
                mnv = jnp.minimum(mnv, vs[k])
                mxv = jnp.maximum(mxv, vs[k])
            for s in (1, 2, 4, 8):
                idx = lax.bitwise_xor(lanes, s)
                mnv = jnp.minimum(mnv, _permute(mnv, idx))
                mxv = jnp.maximum(mxv, _permute(mxv, idx))
            scale = jnp.maximum((mxv - mnv) * (1.0 / MAX_INT), 1e-05)
            inv = 1.0 / scale
            for k in range(8):
                q = ((vs[k] - mnv) * inv + RTNE_MAGIC) - RTNE_MAGIC
                out_v[r, k * 16:(k + 1) * 16] = q * scale + mnv
            return carry2

        lax.fori_loop(0, CHUNK, row_body, 0)
        pltpu.sync_copy(out_v, o_hbm.at[pl.ds(start, CHUNK)])
        return carry

    lax.fori_loop(0, n_chunks, chunk_body, 0)


def kernel(tensor):
    bs, num_heads, seqlen, head_dim = tensor.shape
    rows = bs * num_heads * seqlen
    x = tensor.reshape(rows, head_dim)
    mesh = plsc.VectorSubcoreMesh(core_axis_name="c", subcore_axis_name="s")
    fn = pl.kernel(
        _sc_body,
        mesh=mesh,
        out_type=jax.ShapeDtypeStruct((rows, head_dim), tensor.dtype),
        scratch_types=[
            pltpu.VMEM((CHUNK, head_dim), jnp.float32),
            pltpu.VMEM((CHUNK, head_dim), jnp.float32),
            pltpu.SemaphoreType.DMA,
            pltpu.SemaphoreType.DMA,
        ],
    )
    out = fn(x)
    return out.reshape(bs, num_heads, seqlen, head_dim)


# speedup vs baseline: 2.1607x; 2.1607x over previous
import functools

import jax
import jax.numpy as jnp
from jax import lax
from jax.experimental import pallas as pl
from jax.experimental.pallas import tpu as pltpu
from jax.experimental.pallas import tpu_sc as plsc

MAX_INT = 15.0
RTNE_MAGIC = 12582912.0
NW = 32
CHUNK = 128
BLOCK_ROWS = 16384
CHUNK_ROWS = 128
SC_FRACTION_NUM = 1
SC_FRACTION_DEN = 4

_GDN = lax.GatherDimensionNumbers(
    offset_dims=(), collapsed_slice_dims=(0,), start_index_map=(0,)
)


def _permute(v, idx):
    return lax.gather(
        v,
        idx[:, None],
        dimension_numbers=_GDN,
        slice_sizes=(1,),
        mode=lax.GatherScatterMode.PROMISE_IN_BOUNDS,
    )


def _sc_tail_kernel(x, tail_rows):
    rows = x.shape[0]
    head_rows = rows - tail_rows
    rows_per_w = tail_rows // NW
    n_chunks = rows_per_w // CHUNK

    def body(x_hbm, o_hbm, in_v, out_v, sem_in, sem_out):
        wid = lax.axis_index("s") * 2 + lax.axis_index("c")
        base = head_rows + wid * rows_per_w
        lanes = lax.iota(jnp.int32, 16)

        def chunk_body(ci, carry):
            start = base + ci * CHUNK
            pltpu.sync_copy(x_hbm.at[pl.ds(start, CHUNK)], in_v)

            def row_body(r, carry2):
                vs = [in_v[r, k * 16:(k + 1) * 16] for k in range(8)]
                mnv = vs[0]
                mxv = vs[0]
                for k in range(1, 8):
                    mnv = jnp.minimum(mnv, vs[k])
                    mxv = jnp.maximum(mxv, vs[k])
                for s in (1, 2, 4, 8):
                    idx = lax.bitwise_xor(lanes, s)
                    mnv = jnp.minimum(mnv, _permute(mnv, idx))
                    mxv = jnp.maximum(mxv, _permute(mxv, idx))
                scale = jnp.maximum((mxv - mnv) * (1.0 / MAX_INT), 1e-05)
                inv = 1.0 / scale
                for k in range(8):
                    q = ((vs[k] - mnv) * inv + RTNE_MAGIC) - RTNE_MAGIC
                    out_v[r, k * 16:(k + 1) * 16] = q * scale + mnv
                return carry2

            lax.fori_loop(0, CHUNK, row_body, 0)
            pltpu.sync_copy(out_v, o_hbm.at[pl.ds(ci * CHUNK + wid * rows_per_w, CHUNK)])
            return carry

        lax.fori_loop(0, n_chunks, chunk_body, 0)

    mesh = plsc.VectorSubcoreMesh(core_axis_name="c", subcore_axis_name="s")
    fn = pl.kernel(
        body,
        mesh=mesh,
        out_type=jax.ShapeDtypeStruct((tail_rows, x.shape[1]), x.dtype),
        scratch_types=[
            pltpu.VMEM((CHUNK, x.shape[1]), jnp.float32),
            pltpu.VMEM((CHUNK, x.shape[1]), jnp.float32),
            pltpu.SemaphoreType.DMA,
            pltpu.SemaphoreType.DMA,
        ],
    )
    return fn(x)


def _tc_fq_kernel(x_ref, o_ref):
    for i in range(BLOCK_ROWS // CHUNK_ROWS):
        xc = x_ref[i * CHUNK_ROWS:(i + 1) * CHUNK_ROWS, :]
        mn = jnp.min(xc, axis=-1, keepdims=True)
        mx = jnp.max(xc, axis=-1, keepdims=True)
        scale = jnp.maximum((mx - mn) * (1.0 / MAX_INT), 1e-05)
        q = jnp.round((xc - mn) * (1.0 / scale))
        o_ref[i * CHUNK_ROWS:(i + 1) * CHUNK_ROWS, :] = q * scale + mn


def _tc_head_kernel(x, head_rows):
    rows, head_dim = x.shape
    return pl.pallas_call(
        _tc_fq_kernel,
        out_shape=jax.ShapeDtypeStruct((rows, head_dim), x.dtype),
        grid=(head_rows // BLOCK_ROWS,),
        in_specs=[pl.BlockSpec((BLOCK_ROWS, head_dim), lambda i: (i, 0))],
        out_specs=pl.BlockSpec((BLOCK_ROWS, head_dim), lambda i: (i, 0)),
    )(x)


def kernel(tensor):
    bs, num_heads, seqlen, head_dim = tensor.shape
    rows = bs * num_heads * seqlen
    x = tensor.reshape(rows, head_dim)
    tail_rows = rows * SC_FRACTION_NUM // SC_FRACTION_DEN
    head_rows = rows - tail_rows
    sc_out = _sc_tail_kernel(x, tail_rows)
    tc_out = _tc_head_kernel(x, head_rows)
    out = lax.dynamic_update_slice(tc_out, sc_out, (head_rows, 0))
    return out.reshape(bs, num_heads, seqlen, head_dim)


# speedup vs baseline: 3.0682x; 1.4200x over previous
import jax
import jax.numpy as jnp
from jax.experimental import pallas as pl

MAX_INT = 15.0
BLOCK_ROWS = 16384
CHUNK_ROWS = 32
PAIR = 2


def _fq_chunk(x_ref, o_ref, i):
    xc = x_ref[i * CHUNK_ROWS:(i + 1) * CHUNK_ROWS, :]
    mn = jnp.min(xc, axis=-1, keepdims=True)
    mx = jnp.max(xc, axis=-1, keepdims=True)
    scale = jnp.maximum((mx - mn) * (1.0 / MAX_INT), 1e-05)
    q = jnp.round((xc - mn) * (1.0 / scale))
    o_ref[i * CHUNK_ROWS:(i + 1) * CHUNK_ROWS, :] = q * scale + mn


def _fq_kernel(x_ref, o_ref):
    for i in range(BLOCK_ROWS // CHUNK_ROWS):
        _fq_chunk(x_ref, o_ref, i)


def kernel(tensor):
    bs, num_heads, seqlen, head_dim = tensor.shape
    rows = bs * num_heads * seqlen
    x = tensor.reshape(rows, head_dim)
    out = pl.pallas_call(
        _fq_kernel,
        out_shape=jax.ShapeDtypeStruct((rows, head_dim), tensor.dtype),
        grid=(rows // BLOCK_ROWS,),
        in_specs=[pl.BlockSpec((BLOCK_ROWS, head_dim), lambda i: (i, 0))],
        out_specs=pl.BlockSpec((BLOCK_ROWS, head_dim), lambda i: (i, 0)),
    )(x)
    return out.reshape(bs, num_heads, seqlen, head_dim)
